# baseline (speedup 1.0000x reference)
import functools

import jax
import jax.numpy as jnp
from jax import lax
from jax.experimental import pallas as pl
from jax.experimental.pallas import tpu as pltpu
from jax.experimental.pallas import tpu_sc as plsc

N = 10000
E = 160000
D = 256
R = 8
S = R * N
OUTD = 7
NPAIRS = 100000
BATCH = 16384

NC = 2
NS = 16
LANES = 16

S_PAD = 81920
E_PAD = 163840
TRASH = S_PAD - 1
EPT_A = E_PAD // (NC * NS)
NBA = EPT_A // 128
EPT_D = E_PAD // NS
CH = 4096
NPASS = S_PAD // (CH * NC)
STRIPE = CH // NS
SPT = S_PAD // NS

_mesh = plsc.VectorSubcoreMesh(core_axis_name="c", subcore_axis_name="s")
_sc_params = pltpu.CompilerParams(needs_layout_passes=False)


BT = 64
PLAN_LEN = 163840 // NS + 10 * BT


@functools.partial(
    pl.kernel,
    out_type=[jax.ShapeDtypeStruct((NC * NS, S_PAD), jnp.float32),
              jax.ShapeDtypeStruct((NC * NS, S_PAD), jnp.float32),
              jax.ShapeDtypeStruct((NC, NS, PLAN_LEN), jnp.int32),
              jax.ShapeDtypeStruct((NC, NS, PLAN_LEN), jnp.int32),
              jax.ShapeDtypeStruct((NC, NS, 128), jnp.int32)],
    mesh=_mesh,
    compiler_params=_sc_params,
    scratch_types=[
        pltpu.VMEM((EPT_A,), jnp.int32),
        pltpu.VMEM((S_PAD,), jnp.float32),
        pltpu.VMEM((EPT_D,), jnp.int32),
        pltpu.VMEM((EPT_D,), jnp.int32),
        pltpu.VMEM((PLAN_LEN,), jnp.int32),
        pltpu.VMEM((PLAN_LEN,), jnp.int32),
    ],
)
def _sc_degrees(oslot_hbm, islot_hbm, dego_hbm, degi_hbm,
                civ_hbm, cgv_hbm, cnt_hbm,
                slots_v, hist_v, isv, osv, civ_v, cgv_v):
    c = lax.axis_index("c")
    s = lax.axis_index("s")
    w = c * NS + s
    iota = lax.iota(jnp.int32, LANES)
    ones = jnp.ones((LANES,), jnp.float32)
    zf = jnp.zeros((LANES,), jnp.float32)

    for pi in range(2):
        slot_hbm = (oslot_hbm, islot_hbm)[pi]
        out_hbm = (dego_hbm, degi_hbm)[pi]

        def zero_body(i, _):
            hist_v[pl.ds(i * LANES, LANES)] = zf
            return 0
        lax.fori_loop(0, S_PAD // LANES, zero_body, 0)

        pltpu.sync_copy(slot_hbm.at[pl.ds(w * EPT_A, EPT_A)], slots_v)

        def hg_body(g, _):
            vec = slots_v[pl.ds(g * LANES, LANES)]
            plsc.addupdate_scatter(hist_v, [vec], ones)
            return 0
        lax.fori_loop(0, EPT_A // LANES, hg_body, 0)

        pltpu.sync_copy(hist_v, out_hbm.at[w])

    pltpu.sync_copy(islot_hbm.at[pl.ds(s * EPT_D, EPT_D)], isv)
    pltpu.sync_copy(oslot_hbm.at[pl.ds(s * EPT_D, EPT_D)], osv)
    off = jnp.int32(0)
    cvec = jnp.zeros((LANES,), jnp.int32)
    for p in range(NPASS):
        lo = (p * NC + c) * CH
        hi = lo + CH

        def cmp_body(g, o):
            iv = isv[pl.ds(g * LANES, LANES)]
            ov = osv[pl.ds(g * LANES, LANES)]
            m = (iv >= lo) & (iv < hi)
            plsc.store_compressed(civ_v.at[pl.ds(o, LANES)], iv, mask=m)
            plsc.store_compressed(cgv_v.at[pl.ds(o, LANES)], ov, mask=m)
            cnt = plsc.all_reduce_population_count(m)
            return o + cnt[0]
        off_end = lax.fori_loop(0, EPT_D // LANES, cmp_body, off)

        for t4 in range(BT // LANES):
            civ_v[pl.ds(off_end + t4 * LANES, LANES)] = (
                jnp.full((LANES,), CH, jnp.int32) + lo)
            cgv_v[pl.ds(off_end + t4 * LANES, LANES)] = (
                jnp.zeros((LANES,), jnp.int32))
        nb = (off_end - off + BT - 1) // BT
        cvec = jnp.where(iota == p, nb, cvec)
        off = off + nb * BT

    pltpu.sync_copy(civ_v, civ_hbm.at[c, s])
    pltpu.sync_copy(cgv_v, cgv_hbm.at[c, s])
    slots_v[pl.ds(0, LANES)] = cvec
    for zi in range(1, 8):
        slots_v[pl.ds(zi * LANES, LANES)] = jnp.zeros((LANES,), jnp.int32)
    pltpu.sync_copy(slots_v.at[pl.ds(0, 128)], cnt_hbm.at[c, s])


@functools.partial(
    pl.kernel,
    out_type=jax.ShapeDtypeStruct((2 * S_PAD, 128), jnp.float32),
    mesh=_mesh,
    compiler_params=_sc_params,
    scratch_types=[
        pltpu.VMEM((PLAN_LEN,), jnp.int32),
        pltpu.VMEM((PLAN_LEN,), jnp.int32),
        pltpu.VMEM((128,), jnp.int32),
        pltpu.VMEM((2 * BT,), jnp.int32),
        pltpu.VMEM((2 * BT,), jnp.int32),
        pltpu.VMEM((2, 2 * BT), jnp.int32),
        pltpu.VMEM((2, 2 * BT), jnp.int32),
        pltpu.VMEM((2 * BT, 128), jnp.float32),
        pltpu.VMEM((2 * BT, 128), jnp.float32),
        pltpu.VMEM((64, 128), jnp.float32),
        pltpu.VMEM_SHARED((2 * (CH + LANES), 128), jnp.float32),
    ] + [pltpu.SemaphoreType.DMA] * 5,
)
def _sc_edge_pass(hs_hbm, civ_hbm, cgv_hbm, cnt_hbm, agg_hbm,
                  civ_v, cgv_v, cnt_v, gstg0, gstg1,
                  sstg0, sstg1, buf0, buf1, zbuf, chunk_sh,
                  gsem0, gsem1, ssem0, ssem1, zsem):
    c = lax.axis_index("c")
    s = lax.axis_index("s")
    gstg = (gstg0, gstg1)
    sstg = (sstg0, sstg1)
    bufs = (buf0, buf1)
    gsem = (gsem0, gsem1)
    ssem = (ssem0, ssem1)
    zf = jnp.zeros((LANES,), jnp.float32)

    pltpu.sync_copy(civ_hbm.at[c, s], civ_v)
    pltpu.sync_copy(cgv_hbm.at[c, s], cgv_v)
    pltpu.sync_copy(cnt_hbm.at[c, s], cnt_v)

    def zb_body(i, _):
        for j in range(128 // LANES):
            zbuf[i, pl.ds(j * LANES, LANES)] = zf
        return 0
    lax.fori_loop(0, 64, zb_body, 0)

    def wait_scat(k):
        pltpu.make_async_copy(bufs[k], chunk_sh.at[pl.ds(0, 2 * BT)],
                              ssem[k]).wait()

    off = jnp.int32(0)
    for p in range(NPASS):
        lo = (p * NC + c) * CH
        for z in range(2 * STRIPE // 64):
            pltpu.async_copy(zbuf,
                             chunk_sh.at[pl.ds(2 * s * STRIPE + z * 64, 64)],
                             zsem)
        for z in range(2 * STRIPE // 64):
            pltpu.make_async_copy(zbuf, chunk_sh.at[pl.ds(0, 64)],
                                  zsem).wait()
        plsc.subcore_barrier()

        nb = cnt_v[pl.ds(0, LANES)][p]
        base_p = off

        def stage_fire(b, k):
            for q in range(BT // LANES):
                iv = civ_v[pl.ds(base_p + b * BT + q * LANES, LANES)]
                ov = cgv_v[pl.ds(base_p + b * BT + q * LANES, LANES)]
                gi = ov * 2
                si = (iv - lo) * 2
                gstg[k][pl.ds(q * LANES, LANES)] = gi
                gstg[k][pl.ds(BT + q * LANES, LANES)] = gi + 1
                sstg[k][0, pl.ds(q * LANES, LANES)] = si
                sstg[k][0, pl.ds(BT + q * LANES, LANES)] = si + 1
            pltpu.async_copy(hs_hbm.at[gstg[k]], bufs[k], gsem[k])

        def drain_scat(k):
            pltpu.make_async_copy(hs_hbm.at[pl.ds(0, 2 * BT)], bufs[k],
                                  gsem[k]).wait()
            pltpu.async_copy(bufs[k], chunk_sh.at[sstg[k].at[0]], ssem[k],
                             add=True)

        def gs_body(b2, _):
            for k in range(2):
                b = b2 * 2 + k

                @pl.when(b < nb)
                def _():
                    @pl.when(b >= 2)
                    def _():
                        wait_scat(k)
                    stage_fire(b, k)
            for k in range(2):
                b = b2 * 2 + k

                @pl.when(b < nb)
                def _():
                    drain_scat(k)
            return 0
        lax.fori_loop(0, (nb + 1) // 2, gs_body, 0)
        for k in range(2):
            @pl.when(nb > k)
            def _():
                wait_scat(k)
        plsc.subcore_barrier()

        pltpu.sync_copy(chunk_sh.at[pl.ds(2 * s * STRIPE, 2 * STRIPE)],
                        agg_hbm.at[pl.ds(2 * (lo + s * STRIPE), 2 * STRIPE)])
        off = off + nb * BT


PPT = BATCH // (NC * NS)
QS = NPAIRS // 4


@functools.partial(
    pl.kernel,
    out_type=jax.ShapeDtypeStruct((BATCH, D), jnp.float32),
    mesh=_mesh,
    compiler_params=_sc_params,
    scratch_types=[
        pltpu.VMEM((PPT,), jnp.int32),
        pltpu.VMEM((PPT,), jnp.int32),
        pltpu.VMEM((PPT,), jnp.int32),
        pltpu.VMEM((QS,), jnp.int32),
        pltpu.VMEM((128, D), jnp.float32),
        pltpu.VMEM((128, D), jnp.float32),
        pltpu.SemaphoreType.DMA,
        pltpu.SemaphoreType.DMA,
    ],
)
def _sc_pairs(hx_hbm, ppia_hbm, ppib_hbm, idx_hbm, p_hbm,
              idxv, aidx, bidx, part, abuf, bbuf, sema, semb):
    c = lax.axis_index("c")
    s = lax.axis_index("s")
    base = (c * NS + s) * PPT

    pltpu.sync_copy(idx_hbm.at[pl.ds(base, PPT)], idxv)

    for col in range(2):
        src_hbm = (ppia_hbm, ppib_hbm)[col]
        dstbuf = (aidx, bidx)[col]
        for q in range(4):
            pltpu.sync_copy(src_hbm.at[pl.ds(q * QS, QS)], part)

            def gq_body(g, _):
                iv = idxv[pl.ds(g * LANES, LANES)]
                m = (iv >= q * QS) & (iv < (q + 1) * QS)
                liv = jnp.clip(iv - q * QS, 0, QS - 1)
                got = plsc.load_gather(part, [liv])
                old = dstbuf[pl.ds(g * LANES, LANES)]
                dstbuf[pl.ds(g * LANES, LANES)] = jnp.where(m, got, old)
                return 0
            lax.fori_loop(0, PPT // LANES, gq_body, 0)

    for blk in range(PPT // 128):
        pltpu.async_copy(hx_hbm.at[aidx.at[pl.ds(blk * 128, 128)]],
                         abuf, sema)
        pltpu.async_copy(hx_hbm.at[bidx.at[pl.ds(blk * 128, 128)]],
                         bbuf, semb)
        pltpu.make_async_copy(hx_hbm.at[pl.ds(0, 128)], abuf, sema).wait()
        pltpu.make_async_copy(hx_hbm.at[pl.ds(0, 128)], bbuf, semb).wait()

        def mul_body(i, _):
            for j in range(D // LANES):
                abuf[i, pl.ds(j * LANES, LANES)] = (
                    abuf[i, pl.ds(j * LANES, LANES)]
                    * bbuf[i, pl.ds(j * LANES, LANES)])
            return 0
        lax.fori_loop(0, 128, mul_body, 0)
        pltpu.sync_copy(abuf, p_hbm.at[pl.ds(base + blk * 128, 128)])


def _norm_body(do_ref, di_ref, ns_ref, nd_ref):
    t = jnp.maximum(jnp.sum(do_ref[...], axis=0), 1.0)
    ns_ref[...] = lax.rsqrt(t).reshape(64, 128)
    t2 = jnp.maximum(jnp.sum(di_ref[...], axis=0), 1.0)
    nd_ref[...] = lax.rsqrt(t2).reshape(64, 128)


def _tc_norms(dego, degi):
    ns, nd = pl.pallas_call(
        _norm_body,
        grid=(10,),
        in_specs=[pl.BlockSpec((NC * NS, 8192), lambda i: (0, i)),
                  pl.BlockSpec((NC * NS, 8192), lambda i: (0, i))],
        out_specs=[pl.BlockSpec((64, 128), lambda i: (i, 0)),
                   pl.BlockSpec((64, 128), lambda i: (i, 0))],
        out_shape=[jax.ShapeDtypeStruct((640, 128), jnp.float32),
                   jax.ShapeDtypeStruct((640, 128), jnp.float32)],
    )(dego, degi)
    return ns.reshape(S_PAD)[:S], nd.reshape(S_PAD)[:S]


def _prescale_body(h_ref, ns_ref, hs_ref):
    hs_ref[0] = h_ref[...] * ns_ref[...]


def _tc_prescale(h, ns_col):
    return pl.pallas_call(
        _prescale_body,
        grid=(R, 10),
        in_specs=[pl.BlockSpec((1000, D), lambda r, i: (i, 0)),
                  pl.BlockSpec((1000, 1), lambda r, i: (r * 10 + i, 0))],
        out_specs=pl.BlockSpec((1, 1000, D), lambda r, i: (r, i, 0)),
        out_shape=jax.ShapeDtypeStruct((R, N, D), jnp.float32),
    )(h, ns_col)


def _layer_body(agg_ref, nd_ref, w_ref, b_ref, out_ref, acc_ref):
    r = pl.program_id(1)
    a = agg_ref[0] * nd_ref[...]
    part = jnp.dot(a, w_ref[0], preferred_element_type=jnp.float32)

    @pl.when(r == 0)
    def _():
        acc_ref[...] = part

    @pl.when(r > 0)
    def _():
        acc_ref[...] = acc_ref[...] + part

    @pl.when(r == R - 1)
    def _():
        bsum = jnp.sum(b_ref[...], axis=0)
        out_ref[...] = jnp.maximum(acc_ref[...] + bsum[None, :], 0.0)


def _tc_layer(agg3, nd_col, w_l, b_l):
    return pl.pallas_call(
        _layer_body,
        grid=(10, R),
        in_specs=[pl.BlockSpec((1, 1000, D), lambda i, r: (r, i, 0)),
                  pl.BlockSpec((1000, 1), lambda i, r: (r * 10 + i, 0)),
                  pl.BlockSpec((1, D, D), lambda i, r: (r, 0, 0)),
                  pl.BlockSpec((R, D), lambda i, r: (0, 0))],
        out_specs=pl.BlockSpec((1000, D), lambda i, r: (i, 0)),
        out_shape=jax.ShapeDtypeStruct((N, D), jnp.float32),
        scratch_shapes=[pltpu.VMEM((1000, D), jnp.float32)],
    )(agg3, nd_col, w_l, b_l)


def _linear_body(h_ref, w_ref, b_ref, out_ref):
    out_ref[...] = jnp.maximum(
        jnp.dot(h_ref[...], w_ref[...], preferred_element_type=jnp.float32)
        + b_ref[0][None, :], 0.0)


def _tc_linear(h, w, bvec):
    return pl.pallas_call(
        _linear_body,
        grid=(10,),
        in_specs=[pl.BlockSpec((1000, D), lambda i: (i, 0)),
                  pl.BlockSpec((D, D), lambda i: (0, 0)),
                  pl.BlockSpec((1, D), lambda i: (0, 0))],
        out_specs=pl.BlockSpec((1000, D), lambda i: (i, 0)),
        out_shape=jax.ShapeDtypeStruct((N, D), jnp.float32),
    )(h, w, bvec.reshape(1, D))


def _fc_body(p_ref, w_ref, b_ref, out_ref):
    out_ref[...] = (
        jnp.dot(p_ref[...], w_ref[...], preferred_element_type=jnp.float32)
        + b_ref[0][None, :])


def _tc_fc(p, w, bvec):
    return pl.pallas_call(
        _fc_body,
        grid=(8,),
        in_specs=[pl.BlockSpec((2048, D), lambda i: (i, 0)),
                  pl.BlockSpec((D, OUTD), lambda i: (0, 0)),
                  pl.BlockSpec((1, OUTD), lambda i: (0, 0))],
        out_specs=pl.BlockSpec((2048, OUTD), lambda i: (i, 0)),
        out_shape=jax.ShapeDtypeStruct((BATCH, OUTD), jnp.float32),
    )(p, w, bvec.reshape(1, OUTD))


def kernel(x, edge_index, edge_type, ppi_list, idx, W, b,
           linear_W, linear_b, fc_W, fc_b):
    src = edge_index[0]
    dst = edge_index[1]
    pad_t = jnp.full((E_PAD - E,), TRASH, jnp.int32)
    pad_z = jnp.zeros((E_PAD - E,), jnp.int32)
    oslot = jnp.concatenate([edge_type * N + src, pad_z])
    islot = jnp.concatenate([edge_type * N + dst, pad_t])
    oslot_deg = jnp.concatenate([edge_type * N + src, pad_t])

    dego, degi, civ, cgv, cnt = _sc_degrees(oslot_deg, islot)
    nsf, ndf = _tc_norms(dego, degi)
    ns_col = nsf.reshape(S, 1)
    nd_col = ndf.reshape(S, 1)

    h = x
    for l in range(2):
        hs = _tc_prescale(h, ns_col).reshape(2 * S, 128)
        agg = _sc_edge_pass(hs, civ, cgv, cnt)
        agg = agg.reshape(S_PAD, D)[:S]
        h = _tc_layer(agg.reshape(R, N, D), nd_col, W[l], b[l])

    hx = _tc_linear(h, linear_W, linear_b)
    ppia = ppi_list[:, 0] + 0
    ppib = ppi_list[:, 1] + 0
    p = _sc_pairs(hx, ppia, ppib, idx)
    return _tc_fc(p, fc_W, fc_b)


# speedup vs baseline: 1.3310x; 1.3310x over previous
import functools

import jax
import jax.numpy as jnp
from jax import lax
from jax.experimental import pallas as pl
from jax.experimental.pallas import tpu as pltpu
from jax.experimental.pallas import tpu_sc as plsc

N = 10000
E = 160000
D = 256
R = 8
S = R * N
OUTD = 7
NPAIRS = 100000
BATCH = 16384

NC = 2
NS = 16
LANES = 16

S_PAD = 81920
E_PAD = 163840
TRASH = S_PAD - 1
EPT_A = E_PAD // (NC * NS)
NBA = EPT_A // 128
EPT_D = E_PAD // NS
CH = 5376
NPASS = -(-S_PAD // (CH * NC))
STRIPE = CH // NS
SPT = S_PAD // NS

_mesh = plsc.VectorSubcoreMesh(core_axis_name="c", subcore_axis_name="s")
_sc_params = pltpu.CompilerParams(needs_layout_passes=False)


BT = 64
PLAN_LEN = 163840 // NS + 8 * LANES


@functools.partial(
    pl.kernel,
    out_type=[jax.ShapeDtypeStruct((NC * NS, S_PAD), jnp.float32),
              jax.ShapeDtypeStruct((NC * NS, S_PAD), jnp.float32),
              jax.ShapeDtypeStruct((NC, NS, PLAN_LEN), jnp.int32),
              jax.ShapeDtypeStruct((NC, NS, PLAN_LEN), jnp.int32),
              jax.ShapeDtypeStruct((NC, NS, 128), jnp.int32)],
    mesh=_mesh,
    compiler_params=_sc_params,
    scratch_types=[
        pltpu.VMEM((EPT_A,), jnp.int32),
        pltpu.VMEM((S_PAD,), jnp.float32),
        pltpu.VMEM((EPT_D,), jnp.int32),
        pltpu.VMEM((EPT_D,), jnp.int32),
        pltpu.VMEM((PLAN_LEN,), jnp.int32),
        pltpu.VMEM((PLAN_LEN,), jnp.int32),
    ],
)
def _sc_degrees(oslot_hbm, islot_hbm, dego_hbm, degi_hbm,
                civ_hbm, cgv_hbm, cnt_hbm,
                slots_v, hist_v, isv, osv, civ_v, cgv_v):
    c = lax.axis_index("c")
    s = lax.axis_index("s")
    w = c * NS + s
    iota = lax.iota(jnp.int32, LANES)
    ones = jnp.ones((LANES,), jnp.float32)
    zf = jnp.zeros((LANES,), jnp.float32)

    for pi in range(2):
        slot_hbm = (oslot_hbm, islot_hbm)[pi]
        out_hbm = (dego_hbm, degi_hbm)[pi]

        def zero_body(i, _):
            hist_v[pl.ds(i * LANES, LANES)] = zf
            return 0
        lax.fori_loop(0, S_PAD // LANES, zero_body, 0)

        pltpu.sync_copy(slot_hbm.at[pl.ds(w * EPT_A, EPT_A)], slots_v)

        def hg_body(g, _):
            vec = slots_v[pl.ds(g * LANES, LANES)]
            plsc.addupdate_scatter(hist_v, [vec], ones)
            return 0
        lax.fori_loop(0, EPT_A // LANES, hg_body, 0)

        pltpu.sync_copy(hist_v, out_hbm.at[w])

    pltpu.sync_copy(islot_hbm.at[pl.ds(s * EPT_D, EPT_D)], isv)
    pltpu.sync_copy(oslot_hbm.at[pl.ds(s * EPT_D, EPT_D)], osv)
    off = jnp.int32(0)
    cvec = jnp.zeros((LANES,), jnp.int32)
    for p in range(NPASS):
        lo = (p * NC + c) * CH
        hi = lo + CH

        def cmp_body(g, o):
            iv = isv[pl.ds(g * LANES, LANES)]
            ov = osv[pl.ds(g * LANES, LANES)]
            m = (iv >= lo) & (iv < hi)
            plsc.store_compressed(civ_v.at[pl.ds(o, LANES)], iv, mask=m)
            plsc.store_compressed(cgv_v.at[pl.ds(o, LANES)], ov, mask=m)
            cnt = plsc.all_reduce_population_count(m)
            return o + cnt[0]
        off_end = lax.fori_loop(0, EPT_D // LANES, cmp_body, off)

        civ_v[pl.ds(off_end, LANES)] = (
            jnp.full((LANES,), CH, jnp.int32) + lo)
        cgv_v[pl.ds(off_end, LANES)] = jnp.zeros((LANES,), jnp.int32)
        nb = (off_end - off + LANES - 1) // LANES
        cvec = jnp.where(iota == p, nb, cvec)
        off = off + nb * LANES

    pltpu.sync_copy(civ_v, civ_hbm.at[c, s])
    pltpu.sync_copy(cgv_v, cgv_hbm.at[c, s])
    slots_v[pl.ds(0, LANES)] = cvec
    for zi in range(1, 8):
        slots_v[pl.ds(zi * LANES, LANES)] = jnp.zeros((LANES,), jnp.int32)
    pltpu.sync_copy(slots_v.at[pl.ds(0, 128)], cnt_hbm.at[c, s])


NSLOT = 4


@functools.partial(
    pl.kernel,
    out_type=jax.ShapeDtypeStruct((2 * NPASS * NC * CH, 128), jnp.float32),
    mesh=_mesh,
    compiler_params=_sc_params,
    scratch_types=[
        pltpu.VMEM((PLAN_LEN,), jnp.int32),
        pltpu.VMEM((PLAN_LEN,), jnp.int32),
        pltpu.VMEM((128,), jnp.int32),
    ] + [pltpu.VMEM((LANES, 128), jnp.float32)] * (2 * NSLOT)
      + [pltpu.VMEM((56, 128), jnp.float32),
         pltpu.VMEM_SHARED((2 * (CH + LANES), 128), jnp.float32),
    ] + [pltpu.SemaphoreType.DMA] * (4 * NSLOT + 1),
)
def _sc_edge_pass(hs_hbm, civ_hbm, cgv_hbm, cnt_hbm, agg_hbm,
                  civ_v, cgv_v, cnt_v,
                  bufA0, bufA1, bufA2, bufA3, bufB0, bufB1, bufB2, bufB3,
                  zbuf, chunk_sh,
                  gsemA0, gsemA1, gsemA2, gsemA3,
                  gsemB0, gsemB1, gsemB2, gsemB3,
                  ssemA0, ssemA1, ssemA2, ssemA3,
                  ssemB0, ssemB1, ssemB2, ssemB3, zsem):
    c = lax.axis_index("c")
    s = lax.axis_index("s")
    bufA = (bufA0, bufA1, bufA2, bufA3)
    bufB = (bufB0, bufB1, bufB2, bufB3)
    gsemA = (gsemA0, gsemA1, gsemA2, gsemA3)
    gsemB = (gsemB0, gsemB1, gsemB2, gsemB3)
    ssemA = (ssemA0, ssemA1, ssemA2, ssemA3)
    ssemB = (ssemB0, ssemB1, ssemB2, ssemB3)
    zf = jnp.zeros((LANES,), jnp.float32)

    pltpu.sync_copy(civ_hbm.at[c, s], civ_v)
    pltpu.sync_copy(cgv_hbm.at[c, s], cgv_v)
    pltpu.sync_copy(cnt_hbm.at[c, s], cnt_v)

    def zb_body(i, _):
        for j in range(128 // LANES):
            zbuf[i, pl.ds(j * LANES, LANES)] = zf
        return 0
    lax.fori_loop(0, 56, zb_body, 0)

    def wait_scat(k):
        pltpu.make_async_copy(bufA[k], chunk_sh.at[pl.ds(0, LANES)],
                              ssemA[k]).wait()
        pltpu.make_async_copy(bufB[k], chunk_sh.at[pl.ds(0, LANES)],
                              ssemB[k]).wait()

    off = jnp.int32(0)
    for p in range(NPASS):
        lo = (p * NC + c) * CH
        for z in range(2 * STRIPE // 56):
            pltpu.async_copy(zbuf,
                             chunk_sh.at[pl.ds(2 * s * STRIPE + z * 56, 56)],
                             zsem)
        for z in range(2 * STRIPE // 56):
            pltpu.make_async_copy(zbuf, chunk_sh.at[pl.ds(0, 56)],
                                  zsem).wait()
        plsc.subcore_barrier()

        nb = cnt_v[pl.ds(0, LANES)][p]
        base_p = off

        def stage_fire(b, k):
            ov = cgv_v[pl.ds(base_p + b * LANES, LANES)]
            gi = ov * 2
            pltpu.async_copy(hs_hbm.at[gi], bufA[k], gsemA[k])
            pltpu.async_copy(hs_hbm.at[gi + 1], bufB[k], gsemB[k])

        def drain_scat(b, k):
            iv = civ_v[pl.ds(base_p + b * LANES, LANES)]
            si = (iv - lo) * 2
            pltpu.make_async_copy(hs_hbm.at[pl.ds(0, LANES)], bufA[k],
                                  gsemA[k]).wait()
            pltpu.async_copy(bufA[k], chunk_sh.at[si], ssemA[k], add=True)
            pltpu.make_async_copy(hs_hbm.at[pl.ds(0, LANES)], bufB[k],
                                  gsemB[k]).wait()
            pltpu.async_copy(bufB[k], chunk_sh.at[si + 1], ssemB[k], add=True)

        def gs_body(b4, _):
            for k in range(NSLOT):
                b = b4 * NSLOT + k

                @pl.when(b < nb)
                def _():
                    @pl.when(b >= NSLOT)
                    def _():
                        wait_scat(k)
                    stage_fire(b, k)
            for k in range(NSLOT):
                b = b4 * NSLOT + k

                @pl.when(b < nb)
                def _():
                    drain_scat(b, k)
            return 0
        lax.fori_loop(0, (nb + NSLOT - 1) // NSLOT, gs_body, 0)
        for k in range(NSLOT):
            @pl.when(nb > k)
            def _():
                wait_scat(k)
        plsc.subcore_barrier()

        pltpu.sync_copy(chunk_sh.at[pl.ds(2 * s * STRIPE, 2 * STRIPE)],
                        agg_hbm.at[pl.ds(2 * (lo + s * STRIPE), 2 * STRIPE)])
        off = off + nb * LANES


PPT = BATCH // (NC * NS)
QS = NPAIRS // 4


@functools.partial(
    pl.kernel,
    out_type=jax.ShapeDtypeStruct((BATCH, D), jnp.float32),
    mesh=_mesh,
    compiler_params=_sc_params,
    scratch_types=[
        pltpu.VMEM((PPT,), jnp.int32),
        pltpu.VMEM((PPT,), jnp.int32),
        pltpu.VMEM((PPT,), jnp.int32),
        pltpu.VMEM((QS,), jnp.int32),
        pltpu.VMEM((128, D), jnp.float32),
        pltpu.VMEM((128, D), jnp.float32),
        pltpu.SemaphoreType.DMA,
        pltpu.SemaphoreType.DMA,
    ],
)
def _sc_pairs(hx_hbm, ppia_hbm, ppib_hbm, idx_hbm, p_hbm,
              idxv, aidx, bidx, part, abuf, bbuf, sema, semb):
    c = lax.axis_index("c")
    s = lax.axis_index("s")
    base = (c * NS + s) * PPT

    pltpu.sync_copy(idx_hbm.at[pl.ds(base, PPT)], idxv)

    for col in range(2):
        src_hbm = (ppia_hbm, ppib_hbm)[col]
        dstbuf = (aidx, bidx)[col]
        for q in range(4):
            pltpu.sync_copy(src_hbm.at[pl.ds(q * QS, QS)], part)

            def gq_body(g, _):
                iv = idxv[pl.ds(g * LANES, LANES)]
                m = (iv >= q * QS) & (iv < (q + 1) * QS)
                liv = jnp.clip(iv - q * QS, 0, QS - 1)
                got = plsc.load_gather(part, [liv])
                old = dstbuf[pl.ds(g * LANES, LANES)]
                dstbuf[pl.ds(g * LANES, LANES)] = jnp.where(m, got, old)
                return 0
            lax.fori_loop(0, PPT // LANES, gq_body, 0)

    for blk in range(PPT // 128):
        pltpu.async_copy(hx_hbm.at[aidx.at[pl.ds(blk * 128, 128)]],
                         abuf, sema)
        pltpu.async_copy(hx_hbm.at[bidx.at[pl.ds(blk * 128, 128)]],
                         bbuf, semb)
        pltpu.make_async_copy(hx_hbm.at[pl.ds(0, 128)], abuf, sema).wait()
        pltpu.make_async_copy(hx_hbm.at[pl.ds(0, 128)], bbuf, semb).wait()

        def mul_body(i, _):
            for j in range(D // LANES):
                abuf[i, pl.ds(j * LANES, LANES)] = (
                    abuf[i, pl.ds(j * LANES, LANES)]
                    * bbuf[i, pl.ds(j * LANES, LANES)])
            return 0
        lax.fori_loop(0, 128, mul_body, 0)
        pltpu.sync_copy(abuf, p_hbm.at[pl.ds(base + blk * 128, 128)])


def _norm_body(do_ref, di_ref, ns_ref, nd_ref):
    t = jnp.maximum(jnp.sum(do_ref[...], axis=0), 1.0)
    ns_ref[...] = lax.rsqrt(t).reshape(64, 128)
    t2 = jnp.maximum(jnp.sum(di_ref[...], axis=0), 1.0)
    nd_ref[...] = lax.rsqrt(t2).reshape(64, 128)


def _tc_norms(dego, degi):
    ns, nd = pl.pallas_call(
        _norm_body,
        grid=(10,),
        in_specs=[pl.BlockSpec((NC * NS, 8192), lambda i: (0, i)),
                  pl.BlockSpec((NC * NS, 8192), lambda i: (0, i))],
        out_specs=[pl.BlockSpec((64, 128), lambda i: (i, 0)),
                   pl.BlockSpec((64, 128), lambda i: (i, 0))],
        out_shape=[jax.ShapeDtypeStruct((640, 128), jnp.float32),
                   jax.ShapeDtypeStruct((640, 128), jnp.float32)],
    )(dego, degi)
    return ns.reshape(S_PAD)[:S], nd.reshape(S_PAD)[:S]


def _prescale_body(h_ref, ns_ref, hs_ref):
    hs_ref[0] = h_ref[...] * ns_ref[...]


def _tc_prescale(h, ns_col):
    return pl.pallas_call(
        _prescale_body,
        grid=(R, 10),
        in_specs=[pl.BlockSpec((1000, D), lambda r, i: (i, 0)),
                  pl.BlockSpec((1000, 1), lambda r, i: (r * 10 + i, 0))],
        out_specs=pl.BlockSpec((1, 1000, D), lambda r, i: (r, i, 0)),
        out_shape=jax.ShapeDtypeStruct((R, N, D), jnp.float32),
    )(h, ns_col)


def _layer_body(agg_ref, nd_ref, w_ref, b_ref, out_ref, acc_ref):
    r = pl.program_id(1)
    a = agg_ref[0] * nd_ref[...]
    part = jnp.dot(a, w_ref[0], preferred_element_type=jnp.float32)

    @pl.when(r == 0)
    def _():
        acc_ref[...] = part

    @pl.when(r > 0)
    def _():
        acc_ref[...] = acc_ref[...] + part

    @pl.when(r == R - 1)
    def _():
        bsum = jnp.sum(b_ref[...], axis=0)
        out_ref[...] = jnp.maximum(acc_ref[...] + bsum[None, :], 0.0)


def _tc_layer(agg3, nd_col, w_l, b_l):
    return pl.pallas_call(
        _layer_body,
        grid=(10, R),
        in_specs=[pl.BlockSpec((1, 1000, D), lambda i, r: (r, i, 0)),
                  pl.BlockSpec((1000, 1), lambda i, r: (r * 10 + i, 0)),
                  pl.BlockSpec((1, D, D), lambda i, r: (r, 0, 0)),
                  pl.BlockSpec((R, D), lambda i, r: (0, 0))],
        out_specs=pl.BlockSpec((1000, D), lambda i, r: (i, 0)),
        out_shape=jax.ShapeDtypeStruct((N, D), jnp.float32),
        scratch_shapes=[pltpu.VMEM((1000, D), jnp.float32)],
    )(agg3, nd_col, w_l, b_l)


def _linear_body(h_ref, w_ref, b_ref, out_ref):
    out_ref[...] = jnp.maximum(
        jnp.dot(h_ref[...], w_ref[...], preferred_element_type=jnp.float32)
        + b_ref[0][None, :], 0.0)


def _tc_linear(h, w, bvec):
    return pl.pallas_call(
        _linear_body,
        grid=(10,),
        in_specs=[pl.BlockSpec((1000, D), lambda i: (i, 0)),
                  pl.BlockSpec((D, D), lambda i: (0, 0)),
                  pl.BlockSpec((1, D), lambda i: (0, 0))],
        out_specs=pl.BlockSpec((1000, D), lambda i: (i, 0)),
        out_shape=jax.ShapeDtypeStruct((N, D), jnp.float32),
    )(h, w, bvec.reshape(1, D))


def _fc_body(p_ref, w_ref, b_ref, out_ref):
    out_ref[...] = (
        jnp.dot(p_ref[...], w_ref[...], preferred_element_type=jnp.float32)
        + b_ref[0][None, :])


def _tc_fc(p, w, bvec):
    return pl.pallas_call(
        _fc_body,
        grid=(8,),
        in_specs=[pl.BlockSpec((2048, D), lambda i: (i, 0)),
                  pl.BlockSpec((D, OUTD), lambda i: (0, 0)),
                  pl.BlockSpec((1, OUTD), lambda i: (0, 0))],
        out_specs=pl.BlockSpec((2048, OUTD), lambda i: (i, 0)),
        out_shape=jax.ShapeDtypeStruct((BATCH, OUTD), jnp.float32),
    )(p, w, bvec.reshape(1, OUTD))


def kernel(x, edge_index, edge_type, ppi_list, idx, W, b,
           linear_W, linear_b, fc_W, fc_b):
    src = edge_index[0]
    dst = edge_index[1]
    pad_t = jnp.full((E_PAD - E,), TRASH, jnp.int32)
    pad_z = jnp.zeros((E_PAD - E,), jnp.int32)
    oslot = jnp.concatenate([edge_type * N + src, pad_z])
    islot = jnp.concatenate([edge_type * N + dst, pad_t])
    oslot_deg = jnp.concatenate([edge_type * N + src, pad_t])

    dego, degi, civ, cgv, cnt = _sc_degrees(oslot_deg, islot)
    nsf, ndf = _tc_norms(dego, degi)
    ns_col = nsf.reshape(S, 1)
    nd_col = ndf.reshape(S, 1)

    h = x
    for l in range(2):
        hs = _tc_prescale(h, ns_col).reshape(2 * S, 128)
        agg = _sc_edge_pass(hs, civ, cgv, cnt)
        agg = agg.reshape(NPASS * NC * CH, D)[:S]
        h = _tc_layer(agg.reshape(R, N, D), nd_col, W[l], b[l])

    hx = _tc_linear(h, linear_W, linear_b)
    ppia = ppi_list[:, 0] + 0
    ppib = ppi_list[:, 1] + 0
    p = _sc_pairs(hx, ppia, ppib, idx)
    return _tc_fc(p, fc_W, fc_b)
